# Initial kernel scaffold; baseline (speedup 1.0000x reference)
import jax
import jax.numpy as jnp
from jax.experimental import pallas as pl


def kernel(tile_coords, locs, source_params, slen, ptile_slen, edge_padding):
    raise NotImplementedError("write your pallas kernel here")


# speedup vs baseline: 37.7276x; 37.7276x over previous
import jax
import jax.numpy as jnp
from jax import lax
from jax.experimental import pallas as pl
from jax.experimental.pallas import tpu as pltpu
from jax.experimental.pallas import tpu_sc as plsc

L = 16
NUM_CORES = 2
NUM_SUBCORES = 16
NW = NUM_CORES * NUM_SUBCORES


def kernel(tile_coords, locs, source_params, slen, ptile_slen, edge_padding):
    B, MS, _ = locs.shape
    NPT = tile_coords.shape[0]
    NSP = source_params.shape[2]
    ROWS = B * NPT
    RPW = ROWS // NW
    CHUNKS = MS // L

    f32 = jnp.float32
    i32 = jnp.int32

    ep = jnp.asarray(edge_padding, f32)
    ps = jnp.asarray(ptile_slen, f32)
    tc = tile_coords.astype(f32)
    ax = tc[:, 0] + (ep - 0.5)
    bx = tc[:, 0] + (ps - ep - 0.5)
    ay = tc[:, 1] + (ep - 0.5)
    by = tc[:, 1] + (ps - ep - 0.5)
    bounds = jnp.stack([ax, bx, ay, by], axis=1).reshape(-1)

    params = jnp.zeros((L,), f32)
    params = params.at[0].set(jnp.asarray(slen, f32) - 1.0)
    params = params.at[1].set(ps - 2.0 * ep)

    locs_t = jnp.transpose(locs, (0, 2, 1))
    sp_t = jnp.transpose(source_params, (0, 2, 1))

    mesh = plsc.VectorSubcoreMesh(
        core_axis_name="c", subcore_axis_name="s",
        num_cores=NUM_CORES, num_subcores=NUM_SUBCORES)

    out_type = (
        jax.ShapeDtypeStruct((ROWS, 2 * MS), f32),
        jax.ShapeDtypeStruct((ROWS, NSP * MS), f32),
        jax.ShapeDtypeStruct((ROWS,), i32),
        jax.ShapeDtypeStruct((ROWS, MS), i32),
    )

    @pl.kernel(
        out_type=out_type,
        mesh=mesh,
        compiler_params=pltpu.CompilerParams(needs_layout_passes=False),
        scratch_types=[
            pltpu.VMEM((MS,), f32),
            pltpu.VMEM((MS,), f32),
            pltpu.VMEM((NSP, MS), f32),
            pltpu.VMEM((RPW * 4 + L,), f32),
            pltpu.VMEM((L,), f32),
            pltpu.VMEM((2 * MS,), f32),
            pltpu.VMEM((NSP * MS,), f32),
            pltpu.VMEM((MS,), i32),
            pltpu.VMEM((RPW,), i32),
        ],
    )
    def sc_encode(locs_hbm, sp_hbm, bounds_hbm, params_hbm,
                  out_xy, out_sp, out_n, out_ison,
                  lux_v, luy_v, spv, bounds_v, params_v,
                  rowxy, rowsp, rowison, nbuf):
        wid = lax.axis_index("s") * NUM_CORES + lax.axis_index("c")
        base_row = wid * RPW
        b = base_row // NPT
        p0 = lax.rem(base_row, NPT)

        pltpu.sync_copy(locs_hbm.at[b, 0], lux_v)
        pltpu.sync_copy(locs_hbm.at[b, 1], luy_v)
        pltpu.sync_copy(sp_hbm.at[b], spv)
        pltpu.sync_copy(bounds_hbm.at[pl.ds(p0 * 4, RPW * 4)],
                        bounds_v.at[pl.ds(0, RPW * 4)])
        pltpu.sync_copy(params_hbm, params_v)

        pv = params_v[...]
        slenm1 = pv[0]
        denom = pv[1]
        zf = jnp.zeros((L,), f32)
        zi = jnp.zeros((L,), i32)
        onesi = jnp.ones((L,), i32)
        lane_iota = lax.iota(i32, L)
        lane0 = lane_iota == 0

        def zsp(t, carry):
            rowsp[pl.ds(t * L, L)] = zf
            return carry
        lax.fori_loop(0, (NSP * MS) // L, zsp, 0)

        def zxy(t, carry):
            rowxy[pl.ds(t * L, L)] = zf
            return carry
        lax.fori_loop(0, (2 * MS) // L, zxy, 0)

        def zison(t, carry):
            rowison[pl.ds(t * L, L)] = zi
            return carry
        lax.fori_loop(0, MS // L, zison, 0)

        def row_body(i, carry):
            bv = bounds_v[pl.ds(i * 4, L)]
            axr = bv[0]
            bxr = bv[1]
            ayr = bv[2]
            byr = bv[3]

            def chunk_body(c, n):
                vx = lux_v[pl.ds(c * L, L)] * slenm1
                vy = luy_v[pl.ds(c * L, L)] * slenm1
                m = ((vx > axr) & (vx < bxr) & (vx != 0.0)
                     & (vy > ayr) & (vy < byr) & (vy != 0.0))
                mi = jnp.where(m, onesi, zi)
                cnt = jnp.sum(mi)

                @pl.when(cnt > 0)
                def _():
                    pos = n + plsc.cumsum(mi) - 1
                    tx = (vx - axr) / denom
                    ty = (vy - ayr) / denom
                    plsc.store_scatter(rowxy, [pos * 2], tx, mask=m)
                    plsc.store_scatter(rowxy, [pos * 2 + 1], ty, mask=m)
                    for f in range(NSP):
                        vf = spv[f, pl.ds(c * L, L)]
                        plsc.store_scatter(rowsp, [pos * NSP + f], vf, mask=m)
                    plsc.store_scatter(rowison, [pos], onesi, mask=m)

                return n + cnt

            n = lax.fori_loop(0, CHUNKS, chunk_body, jnp.asarray(0, i32))
            plsc.store_scatter(nbuf, [jnp.full((L,), i, i32)],
                               jnp.full((L,), n, i32), mask=lane0)

            r = base_row + i
            pltpu.sync_copy(rowxy, out_xy.at[r])
            pltpu.sync_copy(rowsp, out_sp.at[r])
            pltpu.sync_copy(rowison, out_ison.at[r])

            def zx(t, carry):
                rowxy[pl.ds(t * L, L)] = zf
                return carry
            lax.fori_loop(0, (2 * n + L - 1) // L, zx, 0)

            def zs(t, carry):
                rowsp[pl.ds(t * L, L)] = zf
                return carry
            lax.fori_loop(0, (NSP * n + L - 1) // L, zs, 0)

            def zo(t, carry):
                rowison[pl.ds(t * L, L)] = zi
                return carry
            lax.fori_loop(0, (n + L - 1) // L, zo, 0)

            return carry

        lax.fori_loop(0, RPW, row_body, 0)
        pltpu.sync_copy(nbuf, out_n.at[pl.ds(base_row, RPW)])

    out_xy, out_sp, out_n, out_ison = sc_encode(locs_t, sp_t, bounds, params)
    new_locs = out_xy.reshape(ROWS, MS, 2)
    new_sp = out_sp.reshape(ROWS, MS, NSP)
    return new_locs, new_sp, out_n, out_ison


# speedup vs baseline: 41.1942x; 1.0919x over previous
import jax
import jax.numpy as jnp
from jax import lax
from jax.experimental import pallas as pl
from jax.experimental.pallas import tpu as pltpu
from jax.experimental.pallas import tpu_sc as plsc

L = 16
NUM_CORES = 2
NUM_SUBCORES = 16
NW = NUM_CORES * NUM_SUBCORES


def kernel(tile_coords, locs, source_params, slen, ptile_slen, edge_padding):
    B, MS, _ = locs.shape
    NPT = tile_coords.shape[0]
    NSP = source_params.shape[2]
    ROWS = B * NPT
    RPW = ROWS // NW
    CHUNKS = MS // L

    f32 = jnp.float32
    i32 = jnp.int32

    ep = jnp.asarray(edge_padding, f32)
    ps = jnp.asarray(ptile_slen, f32)
    tc = tile_coords.astype(f32)
    ax = tc[:, 0] + (ep - 0.5)
    bx = tc[:, 0] + (ps - ep - 0.5)
    ay = tc[:, 1] + (ep - 0.5)
    by = tc[:, 1] + (ps - ep - 0.5)
    bounds = jnp.stack([ax, bx, ay, by], axis=1).reshape(-1)

    params = jnp.zeros((L,), f32)
    params = params.at[0].set(jnp.asarray(slen, f32) - 1.0)
    params = params.at[1].set(ps - 2.0 * ep)

    locs_t = jnp.transpose(locs, (0, 2, 1))
    sp_t = jnp.transpose(source_params, (0, 2, 1))

    mesh = plsc.VectorSubcoreMesh(
        core_axis_name="c", subcore_axis_name="s",
        num_cores=NUM_CORES, num_subcores=NUM_SUBCORES)

    out_type = (
        jax.ShapeDtypeStruct((ROWS, 2 * MS), f32),
        jax.ShapeDtypeStruct((ROWS, NSP * MS), f32),
        jax.ShapeDtypeStruct((ROWS,), i32),
        jax.ShapeDtypeStruct((ROWS, MS), i32),
    )

    @pl.kernel(
        out_type=out_type,
        mesh=mesh,
        compiler_params=pltpu.CompilerParams(needs_layout_passes=False),
        scratch_types=[
            pltpu.VMEM((MS,), f32),
            pltpu.VMEM((MS,), f32),
            pltpu.VMEM((NSP, MS), f32),
            pltpu.VMEM((RPW * 4 + L,), f32),
            pltpu.VMEM((L,), f32),
            pltpu.VMEM((2 * MS,), f32),
            pltpu.VMEM((2 * MS,), f32),
            pltpu.VMEM((NSP * MS,), f32),
            pltpu.VMEM((NSP * MS,), f32),
            pltpu.VMEM((MS,), i32),
            pltpu.VMEM((MS,), i32),
            pltpu.VMEM((RPW,), i32),
            pltpu.SemaphoreType.DMA,
            pltpu.SemaphoreType.DMA,
        ],
    )
    def sc_encode(locs_hbm, sp_hbm, bounds_hbm, params_hbm,
                  out_xy, out_sp, out_n, out_ison,
                  lux_v, luy_v, spv, bounds_v, params_v,
                  rowxyA, rowxyB, rowspA, rowspB, rowisonA, rowisonB,
                  nbuf, semA, semB):
        wid = lax.axis_index("s") * NUM_CORES + lax.axis_index("c")
        base_row = wid * RPW
        b = base_row // NPT
        p0 = lax.rem(base_row, NPT)

        pltpu.sync_copy(locs_hbm.at[b, 0], lux_v)
        pltpu.sync_copy(locs_hbm.at[b, 1], luy_v)
        pltpu.sync_copy(sp_hbm.at[b], spv)
        pltpu.sync_copy(bounds_hbm.at[pl.ds(p0 * 4, RPW * 4)],
                        bounds_v.at[pl.ds(0, RPW * 4)])
        pltpu.sync_copy(params_hbm, params_v)

        pv = params_v[...]
        slenm1 = pv[0]
        denom = pv[1]
        zf = jnp.zeros((L,), f32)
        zi = jnp.zeros((L,), i32)
        onesi = jnp.ones((L,), i32)

        def presc(t, carry):
            lux_v[pl.ds(t * L, L)] = lux_v[pl.ds(t * L, L)] * slenm1
            luy_v[pl.ds(t * L, L)] = luy_v[pl.ds(t * L, L)] * slenm1
            return carry
        lax.fori_loop(0, CHUNKS, presc, 0)

        for rxy, rsp, ris in ((rowxyA, rowspA, rowisonA),
                              (rowxyB, rowspB, rowisonB)):
            def zsp(t, carry, rsp=rsp):
                rsp[pl.ds(t * L, L)] = zf
                return carry
            lax.fori_loop(0, (NSP * MS) // L, zsp, 0)

            def zxy(t, carry, rxy=rxy):
                rxy[pl.ds(t * L, L)] = zf
                return carry
            lax.fori_loop(0, (2 * MS) // L, zxy, 0)

            def zison(t, carry, ris=ris):
                ris[pl.ds(t * L, L)] = zi
                return carry
            lax.fori_loop(0, MS // L, zison, 0)

        def do_row(i, rxy, rsp, ris, sem, n_prev, t):
            @pl.when(t > 0)
            def _():
                pltpu.make_async_copy(rxy, out_xy.at[0], sem).wait()
                pltpu.make_async_copy(rsp, out_sp.at[0], sem).wait()
                pltpu.make_async_copy(ris, out_ison.at[0], sem).wait()

                def zx(u, carry):
                    rxy[pl.ds(u * L, L)] = zf
                    return carry
                lax.fori_loop(0, (2 * n_prev + L - 1) // L, zx, 0)

                def zs(u, carry):
                    rsp[pl.ds(u * L, L)] = zf
                    return carry
                lax.fori_loop(0, (NSP * n_prev + L - 1) // L, zs, 0)

                def zo(u, carry):
                    ris[pl.ds(u * L, L)] = zi
                    return carry
                lax.fori_loop(0, (n_prev + L - 1) // L, zo, 0)

            bv = bounds_v[pl.ds(i * 4, L)]
            axr = bv[0]
            bxr = bv[1]
            ayr = bv[2]
            byr = bv[3]

            def chunk_body(c, n):
                vx = lux_v[pl.ds(c * L, L)]
                vy = luy_v[pl.ds(c * L, L)]
                m = ((vx > axr) & (vx < bxr) & (vx != 0.0)
                     & (vy > ayr) & (vy < byr) & (vy != 0.0))
                cnt = plsc.all_reduce_population_count(m)[0]

                @pl.when(cnt > 0)
                def _():
                    mi = jnp.where(m, onesi, zi)
                    pos = n + plsc.cumsum(mi) - 1
                    tx = (vx - axr) / denom
                    ty = (vy - ayr) / denom
                    plsc.store_scatter(rxy, [pos * 2], tx, mask=m)
                    plsc.store_scatter(rxy, [pos * 2 + 1], ty, mask=m)
                    for f in range(NSP):
                        vf = spv[f, pl.ds(c * L, L)]
                        plsc.store_scatter(rsp, [pos * NSP + f], vf, mask=m)
                    plsc.store_scatter(ris, [pos], onesi, mask=m)

                return n + cnt

            n = lax.fori_loop(0, CHUNKS, chunk_body, jnp.asarray(0, i32),
                              unroll=4)
            plsc.store_scatter(nbuf, [jnp.full((L,), i, i32)],
                               jnp.full((L,), n, i32),
                               mask=lax.iota(i32, L) == 0)

            r = base_row + i
            pltpu.async_copy(rxy, out_xy.at[r], sem)
            pltpu.async_copy(rsp, out_sp.at[r], sem)
            pltpu.async_copy(ris, out_ison.at[r], sem)
            return n

        def pair_body(t, carry):
            nA, nB = carry
            nA = do_row(2 * t, rowxyA, rowspA, rowisonA, semA, nA, t)
            nB = do_row(2 * t + 1, rowxyB, rowspB, rowisonB, semB, nB, t)
            return (nA, nB)

        lax.fori_loop(0, RPW // 2, pair_body,
                      (jnp.asarray(0, i32), jnp.asarray(0, i32)))

        pltpu.make_async_copy(rowxyA, out_xy.at[0], semA).wait()
        pltpu.make_async_copy(rowspA, out_sp.at[0], semA).wait()
        pltpu.make_async_copy(rowisonA, out_ison.at[0], semA).wait()
        pltpu.make_async_copy(rowxyB, out_xy.at[0], semB).wait()
        pltpu.make_async_copy(rowspB, out_sp.at[0], semB).wait()
        pltpu.make_async_copy(rowisonB, out_ison.at[0], semB).wait()

        pltpu.sync_copy(nbuf, out_n.at[pl.ds(base_row, RPW)])

    out_xy, out_sp, out_n, out_ison = sc_encode(locs_t, sp_t, bounds, params)
    new_locs = out_xy.reshape(ROWS, MS, 2)
    new_sp = out_sp.reshape(ROWS, MS, NSP)
    return new_locs, new_sp, out_n, out_ison
